# Initial kernel scaffold; baseline (speedup 1.0000x reference)
import jax
import jax.numpy as jnp
from jax.experimental import pallas as pl


def kernel(feat, W, cu, cv, edge_w, edge_index):
    raise NotImplementedError("write your pallas kernel here")


# speedup vs baseline: 3.4702x; 3.4702x over previous
import functools

import jax
import jax.numpy as jnp
from jax import lax
from jax.experimental import pallas as pl
from jax.experimental.pallas import tpu as pltpu
from jax.experimental.pallas import tpu_sc as plsc

N = 10000
E = 320000
D = 128

NC = 2
NS = 16
CPC = D // NC
NQ = CPC // 16

CW = 80
ROWS_T = E // (NS * CW)
SB = 50
NSB = ROWS_T // SB

RPW = 640
RPW_LAST = N - (NS - 1) * RPW
NB = 80


def _mm_body(feat_ref, cu_ref, w_ref, out_ref):
    x = feat_ref[...] * cu_ref[...]
    out_ref[0] = jnp.dot(x, w_ref[0], preferred_element_type=jnp.float32)


def _matmul(feat, cu, W):
    BLK = 2000
    w2 = W.reshape(D, NC, CPC).transpose(1, 0, 2)
    return pl.pallas_call(
        _mm_body,
        grid=(NC, N // BLK),
        in_specs=[
            pl.BlockSpec((BLK, D), lambda c, i: (i, 0)),
            pl.BlockSpec((BLK, 1), lambda c, i: (i, 0)),
            pl.BlockSpec((1, D, CPC), lambda c, i: (c, 0, 0)),
        ],
        out_specs=pl.BlockSpec((1, BLK, CPC), lambda c, i: (c, i, 0)),
        out_shape=jax.ShapeDtypeStruct((NC, N, CPC), jnp.float32),
    )(feat, cu, w2)


def _sc_body(h_hbm, src_hbm, dst_hbm, w_hbm, cv_hbm, out_hbm,
             table_sh, acc_sh, src_v, dst_v, w_v, rows_v, node_v, cv_v, sem):
    c = lax.axis_index("c")
    s = lax.axis_index("s")
    r0 = s * RPW
    nrows = jnp.where(s < NS - 1, RPW, RPW_LAST)
    nblk = jnp.where(s < NS - 1, RPW // NB, RPW_LAST // NB)

    zero16 = jnp.zeros((16,), jnp.float32)

    def zero_body(i, _):
        for q in range(NQ):
            node_v[i, pl.ds(q * 16, 16)] = zero16
        return 0
    lax.fori_loop(0, NB, zero_body, 0)

    def zcopy_body(b, _):
        pltpu.sync_copy(node_v, acc_sh.at[pl.ds(r0 + b * NB, NB)])
        return 0
    lax.fori_loop(0, nblk, zcopy_body, 0)
    pltpu.sync_copy(h_hbm.at[c, pl.ds(r0, nrows)],
                    table_sh.at[pl.ds(r0, nrows)])

    plsc.subcore_barrier()

    def sb_body(sb, _):
        pltpu.sync_copy(src_hbm.at[s, pl.ds(sb * SB, SB)], src_v)
        pltpu.sync_copy(dst_hbm.at[s, pl.ds(sb * SB, SB)], dst_v)
        pltpu.sync_copy(w_hbm.at[s, pl.ds(sb * SB, SB)], w_v)

        def chunk_body(j, _):
            pltpu.async_copy(table_sh.at[src_v.at[j]], rows_v, sem).wait()

            def group_body(g, _):
                wvec = w_v[j, pl.ds(g * 16, 16)]
                for q in range(16):
                    wq = wvec[q]
                    i = g * 16 + q
                    for r in range(NQ):
                        sl = pl.ds(r * 16, 16)
                        rows_v[i, sl] = rows_v[i, sl] * wq
                return 0
            lax.fori_loop(0, CW // 16, group_body, 0)

            pltpu.sync_copy(rows_v, acc_sh.at[dst_v.at[j]], add=True)
            return 0
        lax.fori_loop(0, SB, chunk_body, 0)
        return 0
    lax.fori_loop(0, NSB, sb_body, 0)

    plsc.subcore_barrier()

    def out_blk_body(b, _):
        rb = r0 + b * NB
        pltpu.sync_copy(acc_sh.at[pl.ds(rb, NB)], node_v)
        pltpu.sync_copy(cv_hbm.at[pl.ds(rb, NB)], cv_v)

        def out_body(g, _):
            cvec = cv_v[pl.ds(g * 16, 16)]
            for q in range(16):
                cq = cvec[q]
                i = g * 16 + q
                for r in range(NQ):
                    sl = pl.ds(r * 16, 16)
                    node_v[i, sl] = node_v[i, sl] * cq
            return 0
        lax.fori_loop(0, NB // 16, out_body, 0)
        pltpu.sync_copy(node_v, out_hbm.at[c, pl.ds(rb, NB)])
        return 0
    lax.fori_loop(0, nblk, out_blk_body, 0)


@jax.jit
def kernel(feat, W, cu, cv, edge_w, edge_index):
    h = _matmul(feat, cu, W)

    src = edge_index[0].reshape(NS, ROWS_T, CW)
    dst = edge_index[1].reshape(NS, ROWS_T, CW)
    ew = edge_w.reshape(NS, ROWS_T, CW)
    cv1 = cv.reshape(N)

    mesh = plsc.VectorSubcoreMesh(core_axis_name="c", subcore_axis_name="s")
    sc_fn = pl.kernel(
        _sc_body,
        out_type=jax.ShapeDtypeStruct((NC, N, CPC), jnp.float32),
        mesh=mesh,
        compiler_params=pltpu.CompilerParams(use_tc_tiling_on_sc=False),
        scratch_types=[
            pltpu.VMEM_SHARED((N, CPC), jnp.float32),
            pltpu.VMEM_SHARED((N, CPC), jnp.float32),
            pltpu.VMEM((SB, CW), jnp.int32),
            pltpu.VMEM((SB, CW), jnp.int32),
            pltpu.VMEM((SB, CW), jnp.float32),
            pltpu.VMEM((CW, CPC), jnp.float32),
            pltpu.VMEM((NB, CPC), jnp.float32),
            pltpu.VMEM((NB,), jnp.float32),
            pltpu.SemaphoreType.DMA,
        ],
    )
    out2 = sc_fn(h, src, dst, ew, cv1)
    return jnp.concatenate([out2[0], out2[1]], axis=1)


# speedup vs baseline: 8.1613x; 2.3518x over previous
import functools

import jax
import jax.numpy as jnp
from jax import lax
from jax.experimental import pallas as pl
from jax.experimental.pallas import tpu as pltpu
from jax.experimental.pallas import tpu_sc as plsc

N = 10000
E = 320000
D = 128

NC = 2
NS = 16
CPC = D // NC
NQ = CPC // 16

CW = 128
ROWS_T = 160
EP = NS * ROWS_T * CW
SB = 80
NSB = ROWS_T // SB

RPW = 640
RPW_LAST = N - (NS - 1) * RPW
NB = 80


def _mm_body(feat_ref, cu_ref, w_ref, out_ref):
    x = feat_ref[...] * cu_ref[...]
    out_ref[0] = jnp.dot(x, w_ref[0], preferred_element_type=jnp.float32)


def _matmul(feat, cu, W):
    BLK = 2000
    w2 = W.reshape(D, NC, CPC).transpose(1, 0, 2)
    return pl.pallas_call(
        _mm_body,
        grid=(NC, N // BLK),
        in_specs=[
            pl.BlockSpec((BLK, D), lambda c, i: (i, 0)),
            pl.BlockSpec((BLK, 1), lambda c, i: (i, 0)),
            pl.BlockSpec((1, D, CPC), lambda c, i: (c, 0, 0)),
        ],
        out_specs=pl.BlockSpec((1, BLK, CPC), lambda c, i: (c, i, 0)),
        out_shape=jax.ShapeDtypeStruct((NC, N, CPC), jnp.float32),
    )(feat, cu, w2)


def _scale_chunk(w_ref, j, in_ref, out_ref):
    for g in range(CW // 16):
        wvec = w_ref[j, pl.ds(g * 16, 16)]
        for qq in range(0, 16, 4):
            vals = []
            for q in range(qq, qq + 4):
                wq = wvec[q]
                i = g * 16 + q
                for r in range(NQ):
                    vals.append(in_ref[i, pl.ds(r * 16, 16)] * wq)
            k = 0
            for q in range(qq, qq + 4):
                i = g * 16 + q
                for r in range(NQ):
                    out_ref[i, pl.ds(r * 16, 16)] = vals[k]
                    k += 1


def _sc_body(ha_hbm, hb_hbm, src_hbm, dst_hbm, w_hbm, cv_hbm, out_hbm,
             acc_sh, src_v, dst_v, w_v,
             rows0, rows1, mrows0, mrows1, node_v, cv_v,
             sem_g0, sem_g1, sem_s0, sem_s1):
    c = lax.axis_index("c")
    s = lax.axis_index("s")
    r0 = s * RPW
    nrows = jnp.where(s < NS - 1, RPW, RPW_LAST)
    nblk = jnp.where(s < NS - 1, RPW // NB, RPW_LAST // NB)

    rows = (rows0, rows1)
    mrows = (mrows0, mrows1)
    sem_g = (sem_g0, sem_g1)
    sem_s = (sem_s0, sem_s1)

    def gather_start(jj, b):
        @pl.when(c == 0)
        def _():
            pltpu.async_copy(ha_hbm.at[src_v.at[jj]], rows[b], sem_g[b])
        @pl.when(c == 1)
        def _():
            pltpu.async_copy(hb_hbm.at[src_v.at[jj]], rows[b], sem_g[b])

    def gather_wait(jj, b):
        @pl.when(c == 0)
        def _():
            pltpu.make_async_copy(
                ha_hbm.at[src_v.at[jj]], rows[b], sem_g[b]).wait()
        @pl.when(c == 1)
        def _():
            pltpu.make_async_copy(
                hb_hbm.at[src_v.at[jj]], rows[b], sem_g[b]).wait()

    zero16 = jnp.zeros((16,), jnp.float32)

    def zero_body(i, _):
        for q in range(NQ):
            node_v[i, pl.ds(q * 16, 16)] = zero16
        return 0
    lax.fori_loop(0, NB, zero_body, 0)

    def zcopy_body(b, _):
        pltpu.sync_copy(node_v, acc_sh.at[pl.ds(r0 + b * NB, NB)])
        return 0
    lax.fori_loop(0, nblk, zcopy_body, 0)

    plsc.subcore_barrier()

    def sb_body(sb, _):
        pltpu.sync_copy(src_hbm.at[s, pl.ds(sb * SB, SB)], src_v)
        pltpu.sync_copy(dst_hbm.at[s, pl.ds(sb * SB, SB)], dst_v)
        pltpu.sync_copy(w_hbm.at[s, pl.ds(sb * SB, SB)], w_v)

        gather_start(0, 0)

        def pair_body(j0, _):
            for b in range(2):
                j = j0 + b
                gather_wait(j, b)
                if b == 0:
                    gather_start(j + 1, 1)
                else:
                    @pl.when(j0 < SB - 2)
                    def _():
                        gather_start(j + 1, 0)
                @pl.when(j0 >= 2)
                def _():
                    pltpu.make_async_copy(
                        mrows[b], acc_sh.at[dst_v.at[j]], sem_s[b]).wait()
                _scale_chunk(w_v, j, rows[b], mrows[b])
                pltpu.async_copy(
                    mrows[b], acc_sh.at[dst_v.at[j]], sem_s[b], add=True)
            return 0
        lax.fori_loop(0, SB // 2, lambda p, _: pair_body(p * 2, _), 0)

        for b in range(2):
            pltpu.make_async_copy(
                mrows[b], acc_sh.at[dst_v.at[SB - 2 + b]], sem_s[b]).wait()
        return 0
    lax.fori_loop(0, NSB, sb_body, 0)

    plsc.subcore_barrier()

    def out_blk_body(b, _):
        rb = r0 + b * NB
        pltpu.sync_copy(acc_sh.at[pl.ds(rb, NB)], node_v)
        pltpu.sync_copy(cv_hbm.at[pl.ds(rb, NB)], cv_v)

        def out_body(g, _):
            cvec = cv_v[pl.ds(g * 16, 16)]
            for qq in range(0, 16, 4):
                vals = []
                for q in range(qq, qq + 4):
                    cq = cvec[q]
                    i = g * 16 + q
                    for r in range(NQ):
                        vals.append(node_v[i, pl.ds(r * 16, 16)] * cq)
                k = 0
                for q in range(qq, qq + 4):
                    i = g * 16 + q
                    for r in range(NQ):
                        node_v[i, pl.ds(r * 16, 16)] = vals[k]
                        k += 1
            return 0
        lax.fori_loop(0, NB // 16, out_body, 0)
        pltpu.sync_copy(node_v, out_hbm.at[pl.ds(rb, NB), pl.ds(c * CPC, CPC)])
        return 0
    lax.fori_loop(0, nblk, out_blk_body, 0)


@jax.jit
def kernel(feat, W, cu, cv, edge_w, edge_index):
    h = _matmul(feat, cu, W)
    ha = h[0]
    hb = h[1]

    pad = EP - E
    pad_idx = (jnp.arange(pad, dtype=jnp.int32) * 37) % N
    src = jnp.concatenate([edge_index[0], pad_idx]).reshape(NS, ROWS_T, CW)
    dst = jnp.concatenate([edge_index[1], pad_idx]).reshape(NS, ROWS_T, CW)
    ew = jnp.concatenate(
        [edge_w.reshape(E), jnp.zeros((pad,), jnp.float32)]
    ).reshape(NS, ROWS_T, CW)
    cv1 = cv.reshape(N)

    mesh = plsc.VectorSubcoreMesh(core_axis_name="c", subcore_axis_name="s")
    sc_fn = pl.kernel(
        _sc_body,
        out_type=jax.ShapeDtypeStruct((N, D), jnp.float32),
        mesh=mesh,
        compiler_params=pltpu.CompilerParams(use_tc_tiling_on_sc=False),
        scratch_types=[
            pltpu.VMEM_SHARED((N, CPC), jnp.float32),
            pltpu.VMEM((SB, CW), jnp.int32),
            pltpu.VMEM((SB, CW), jnp.int32),
            pltpu.VMEM((SB, CW), jnp.float32),
            pltpu.VMEM((CW, CPC), jnp.float32),
            pltpu.VMEM((CW, CPC), jnp.float32),
            pltpu.VMEM((CW, CPC), jnp.float32),
            pltpu.VMEM((CW, CPC), jnp.float32),
            pltpu.VMEM((NB, CPC), jnp.float32),
            pltpu.VMEM((NB,), jnp.float32),
            pltpu.SemaphoreType.DMA,
            pltpu.SemaphoreType.DMA,
            pltpu.SemaphoreType.DMA,
            pltpu.SemaphoreType.DMA,
        ],
    )
    return sc_fn(ha, hb, src, dst, ew, cv1)
